# baseline (speedup 1.0000x reference)
import functools

import jax
import jax.numpy as jnp
from jax import lax
from jax.experimental import pallas as pl
from jax.experimental.pallas import tpu as pltpu
from jax.experimental.pallas import tpu_sc as plsc

BATCH = 16384
DIM = 64
MARGIN = 1.0
NC = 2
NS = 16
NW = NC * NS
PER_W = BATCH // NW
CHUNK = 128
NCHUNK = PER_W // CHUNK
GROUPS = CHUNK // 16

_EPS = 1e-12


def _rsqrt(x):
    i = lax.bitcast_convert_type(x, jnp.int32)
    i = jnp.int32(0x5F3759DF) - lax.shift_right_arithmetic(i, jnp.int32(1))
    y = lax.bitcast_convert_type(i, jnp.float32)
    for _ in range(3):
        y = y * (1.5 - 0.5 * x * y * y)
    return y


def _body(ph, pt, pr, nh, nt, nr, ent_e, rel_e, ent_t, rel_t, out,
          ph_v, pt_v, pr_v, nh_v, nt_v, nr_v,
          he_p, ht_p, te_p, tt_p, re_p, rt_p,
          he_n, ht_n, te_n, tt_n, re_n, rt_n,
          out_v, sem):
    wid = lax.axis_index("s") * NC + lax.axis_index("c")
    base = wid * PER_W

    for src, dst in ((ph, ph_v), (pt, pt_v), (pr, pr_v),
                     (nh, nh_v), (nt, nt_v), (nr, nr_v)):
        pltpu.sync_copy(src.at[pl.ds(base, PER_W)], dst)

    lanes = lax.iota(jnp.int32, 16)
    acc = jnp.zeros((16,), jnp.float32)

    for c in range(NCHUNK):
        sl = pl.ds(c * CHUNK, CHUNK)
        copies = []
        for tbl, idx_v, buf in (
                (ent_e, ph_v, he_p), (ent_t, ph_v, ht_p),
                (ent_e, pt_v, te_p), (ent_t, pt_v, tt_p),
                (rel_e, pr_v, re_p), (rel_t, pr_v, rt_p),
                (ent_e, nh_v, he_n), (ent_t, nh_v, ht_n),
                (ent_e, nt_v, te_n), (ent_t, nt_v, tt_n),
                (rel_e, nr_v, re_n), (rel_t, nr_v, rt_n)):
            copies.append(pltpu.async_copy(tbl.at[idx_v.at[sl]], buf, sem))
        for cp in copies:
            cp.wait()

        def group_body(g, acc):
            rows = g * 16 + lanes

            def dots(d, carry):
                col = jnp.full((16,), d, jnp.int32)
                he = plsc.load_gather(he_p, [rows, col])
                ht = plsc.load_gather(ht_p, [rows, col])
                te = plsc.load_gather(te_p, [rows, col])
                tt = plsc.load_gather(tt_p, [rows, col])
                rt = plsc.load_gather(rt_p, [rows, col])
                hen = plsc.load_gather(he_n, [rows, col])
                htn = plsc.load_gather(ht_n, [rows, col])
                ten = plsc.load_gather(te_n, [rows, col])
                ttn = plsc.load_gather(tt_n, [rows, col])
                rtn = plsc.load_gather(rt_n, [rows, col])
                (sh, st, shh, stt, srr, shr, str_,
                 sh2, st2, shh2, stt2, srr2, shr2, str2) = carry
                return (sh + he * ht, st + te * tt,
                        shh + he * he, stt + te * te, srr + rt * rt,
                        shr + he * rt, str_ + te * rt,
                        sh2 + hen * htn, st2 + ten * ttn,
                        shh2 + hen * hen, stt2 + ten * ten,
                        srr2 + rtn * rtn, shr2 + hen * rtn,
                        str2 + ten * rtn)

            z = jnp.zeros((16,), jnp.float32)
            (sh, st, shh, stt, srr, shr, str_,
             sh2, st2, shh2, stt2, srr2, shr2, str2) = lax.fori_loop(
                0, DIM, dots, (z,) * 14)

            nh_sq = shh + 2.0 * sh * shr + sh * sh * srr
            nt_sq = stt + 2.0 * st * str_ + st * st * srr
            nh_sq2 = shh2 + 2.0 * sh2 * shr2 + sh2 * sh2 * srr2
            nt_sq2 = stt2 + 2.0 * st2 * str2 + st2 * st2 * srr2
            ah = _rsqrt(jnp.maximum(nh_sq, _EPS))
            at = _rsqrt(jnp.maximum(nt_sq, _EPS))
            ah2 = _rsqrt(jnp.maximum(nh_sq2, _EPS))
            at2 = _rsqrt(jnp.maximum(nt_sq2, _EPS))
            crt = ah * sh - at * st
            crt2 = ah2 * sh2 - at2 * st2

            def dist(d, carry):
                pacc, nacc = carry
                col = jnp.full((16,), d, jnp.int32)
                he = plsc.load_gather(he_p, [rows, col])
                te = plsc.load_gather(te_p, [rows, col])
                rt = plsc.load_gather(rt_p, [rows, col])
                re = plsc.load_gather(re_p, [rows, col])
                hen = plsc.load_gather(he_n, [rows, col])
                ten = plsc.load_gather(te_n, [rows, col])
                rtn = plsc.load_gather(rt_n, [rows, col])
                ren = plsc.load_gather(re_n, [rows, col])
                p = ah * he - at * te + crt * rt + re
                n = ah2 * hen - at2 * ten + crt2 * rtn + ren
                return pacc + jnp.abs(p), nacc + jnp.abs(n)

            pos, neg = lax.fori_loop(0, DIM, dist, (z, z))
            return acc + jnp.maximum(pos - neg + MARGIN, 0.0)

        acc = lax.fori_loop(0, GROUPS, group_body, acc)

    out_v[...] = acc
    pltpu.sync_copy(out_v, out.at[wid])


def kernel(x, ent_embeddings, rel_embeddings, ent_transfer, rel_transfer):
    cols = tuple(x[:, j] for j in range(6))
    mesh = plsc.VectorSubcoreMesh(core_axis_name="c", subcore_axis_name="s")
    row_buf = pltpu.VMEM((CHUNK, DIM), jnp.float32)
    idx_buf = pltpu.VMEM((PER_W,), jnp.int32)
    partials = pl.kernel(
        _body,
        out_type=jax.ShapeDtypeStruct((NW, 16), jnp.float32),
        mesh=mesh,
        scratch_types=[idx_buf] * 6 + [row_buf] * 12
        + [pltpu.VMEM((16,), jnp.float32), pltpu.SemaphoreType.DMA],
        compiler_params=pltpu.CompilerParams(
            needs_layout_passes=False, use_tc_tiling_on_sc=False),
    )(*cols, ent_embeddings, rel_embeddings, ent_transfer, rel_transfer)
    return jnp.sum(partials)
